# baseline (speedup 1.0000x reference)
import jax
import jax.numpy as jnp
from jax import lax
from jax.experimental import pallas as pl
from jax.experimental.pallas import tpu as pltpu, tpu_sc as plsc

TEMP_INV = 10.0
N = 2048
D = 128
NC = 2
NS = 16
NW = NC * NS
ROWS_PER_TEC = N // NW
NBINS = 13

INV3 = -1431655765
LIM3 = 0x55555555


def _valuation(m):
    v = jnp.zeros((16,), jnp.float32)
    for _ in range(NBINS):
        t = m * INV3
        div = (m > 0) & (t >= 0) & (t <= LIM3)
        v = v + jnp.where(div, 1.0, 0.0)
        m = jnp.where(div, t, m)
    return v


def _tc_radii_body(z_ref, ep_ref, en_ref):
    x = z_ref[...]
    q = jnp.sqrt(jnp.sum(x * x, axis=-1)) * TEMP_INV
    qs = q - 0.5 * (jnp.max(q) + jnp.min(q))
    qs = jnp.clip(qs, -87.0, 87.0)
    ep_ref[...] = jnp.exp(qs)
    en_ref[...] = jnp.exp(-qs)


def _sc_body(ep_hbm, en_hbm, bidx_hbm, out_hbm, etab, enb, idxb, stage):
    cid = lax.axis_index("c")
    sid = lax.axis_index("s")
    wid = cid * NS + sid

    pltpu.sync_copy(ep_hbm, etab)
    pltpu.sync_copy(en_hbm.at[pl.ds(wid * ROWS_PER_TEC, ROWS_PER_TEC)], enb)
    pltpu.sync_copy(bidx_hbm.at[pl.ds(wid * ROWS_PER_TEC, ROWS_PER_TEC)], idxb)

    vvecs = [_valuation(idxb[pl.ds(k * 16, 16)]) for k in range(4)]
    evecs = [enb[pl.ds(k * 16, 16)] for k in range(4)]

    one = jnp.ones((16,), jnp.float32)

    def jstep(t, accs):
        ejv = etab[pl.ds(t * 16, 16)]
        for dj in range(16):
            ej = jnp.full((16,), ejv[dj])
            accs = tuple(
                acc + one / (one + ej * ei)
                for acc, ei in zip(accs, evecs)
            )
        return accs

    zero = jnp.zeros((16,), jnp.float32)
    accs = lax.fori_loop(0, N // 16, jstep, (zero, zero, zero, zero))

    for a in range(NBINS):
        cnt = zero
        for vk in vvecs:
            cnt = cnt + jnp.where(vk == float(a), 1.0, 0.0)
        stage[a, :] = cnt
    bvec = zero
    for vk, acck in zip(vvecs, accs):
        bvec = bvec + vk * acck
    stage[NBINS, :] = bvec
    pltpu.sync_copy(stage, out_hbm.at[wid])


def kernel(z_hyp, batch_indices):
    ep, en = pl.pallas_call(
        _tc_radii_body,
        out_shape=(
            jax.ShapeDtypeStruct((8, N // 8), jnp.float32),
            jax.ShapeDtypeStruct((8, N // 8), jnp.float32),
        ),
    )(z_hyp.reshape(8, N // 8, D))

    mesh = plsc.VectorSubcoreMesh(core_axis_name="c", subcore_axis_name="s")
    out = pl.kernel(
        _sc_body,
        mesh=mesh,
        out_type=jax.ShapeDtypeStruct((NW, NBINS + 1, 16), jnp.float32),
        scratch_types=[
            pltpu.VMEM((N,), jnp.float32),
            pltpu.VMEM((ROWS_PER_TEC,), jnp.float32),
            pltpu.VMEM((ROWS_PER_TEC,), jnp.int32),
            pltpu.VMEM((NBINS + 1, 16), jnp.float32),
        ],
    )(ep.reshape(N), en.reshape(N), batch_indices)

    b_total = jnp.sum(out[:, NBINS, :])
    c = jnp.sum(out[:, :NBINS, :], axis=(0, 2))
    a_idx = jnp.arange(NBINS, dtype=jnp.float32)
    sv = jnp.sum(a_idx * c)
    csq = jnp.sum(c * c)
    a_sum = 0.5 * jnp.sum(c[:, None] * c[None, :]
                          * jnp.abs(a_idx[:, None] - a_idx[None, :]))
    denom = jnp.maximum(float(N * N) - csq, 1.0)
    num = 2.0 * b_total - float(N) * sv + a_sum
    return num / denom


# speedup vs baseline: 1.0338x; 1.0338x over previous
import jax
import jax.numpy as jnp
from jax import lax
from jax.experimental import pallas as pl
from jax.experimental.pallas import tpu as pltpu, tpu_sc as plsc

TEMP_INV = 10.0
N = 2048
D = 128
NC = 2
NS = 16
NW = NC * NS
SC_ROWS = 1024
ROWS_PER_TEC = SC_ROWS // NW
TC_CHUNK = 256
NBINS = 13

INV3 = -1431655765
LIM3 = 0x55555555


def _valuation_sc(m):
    v = jnp.zeros((16,), jnp.float32)
    for _ in range(NBINS):
        t = m * INV3
        div = (m > 0) & (t >= 0) & (t <= LIM3)
        v = v + jnp.where(div, 1.0, 0.0)
        m = jnp.where(div, t, m)
    return v


def _tc_radii_body(z_ref, qs_ref, ep_ref):
    x = z_ref[...]
    q = jnp.sqrt(jnp.sum(x * x, axis=-1)) * TEMP_INV
    qs = q - 0.5 * (jnp.max(q) + jnp.min(q))
    qs = jnp.clip(qs, -87.0, 87.0)
    qs_ref[...] = qs
    ep_ref[...] = jnp.exp(qs)


def _tc_pair_body(qs_ref, ep_ref, bidx_ref, outb_ref, outc_ref):
    en = jnp.exp(-qs_ref[...])
    ep = ep_ref[...].reshape(1, N)
    sig = 1.0 / (1.0 + en.reshape(TC_CHUNK, 1) * ep)
    g = jnp.sum(sig, axis=-1)

    m = bidx_ref[...].reshape(TC_CHUNK)
    v = jnp.zeros((TC_CHUNK,), jnp.float32)
    for _ in range(NBINS):
        div = (m > 0) & (m % 3 == 0)
        v = v + div.astype(jnp.float32)
        m = jnp.where(div, m // 3, m)

    b_part = jnp.sum(v * g)
    lane = lax.broadcasted_iota(jnp.int32, (1, 16), 1)
    cnt = jnp.zeros((1, 16), jnp.float32)
    for a in range(NBINS):
        cnt = cnt + jnp.where(lane == a, jnp.sum(
            jnp.where(v == float(a), 1.0, 0.0)), 0.0)

    @pl.when(pl.program_id(0) == 0)
    def _():
        outb_ref[...] = jnp.zeros((1, 1), jnp.float32)
        outc_ref[...] = jnp.zeros((1, 16), jnp.float32)

    outb_ref[...] += b_part.reshape(1, 1)
    outc_ref[...] += cnt


def _sc_body(ep_hbm, qs_hbm, bidx_hbm, out_hbm, etab, qsb, idxb, stage):
    cid = lax.axis_index("c")
    sid = lax.axis_index("s")
    wid = cid * NS + sid

    pltpu.sync_copy(ep_hbm, etab)
    pltpu.sync_copy(qs_hbm.at[pl.ds(wid * ROWS_PER_TEC, ROWS_PER_TEC)], qsb)
    pltpu.sync_copy(bidx_hbm.at[pl.ds(wid * ROWS_PER_TEC, ROWS_PER_TEC)], idxb)

    vvecs = [_valuation_sc(idxb[pl.ds(k * 16, 16)]) for k in range(2)]
    evecs = [jnp.exp(-qsb[pl.ds(k * 16, 16)]) for k in range(2)]

    one = jnp.ones((16,), jnp.float32)

    def jstep(t, accs):
        ejv = etab[pl.ds(t * 16, 16)]
        for dj in range(16):
            ej = jnp.full((16,), ejv[dj])
            accs = tuple(
                acc + one / (one + ej * ei)
                for acc, ei in zip(accs, evecs)
            )
        return accs

    zero = jnp.zeros((16,), jnp.float32)
    accs = lax.fori_loop(0, N // 16, jstep, (zero, zero))

    for a in range(NBINS):
        cnt = zero
        for vk in vvecs:
            cnt = cnt + jnp.where(vk == float(a), 1.0, 0.0)
        stage[a, :] = cnt
    bvec = zero
    for vk, acck in zip(vvecs, accs):
        bvec = bvec + vk * acck
    stage[NBINS, :] = bvec
    pltpu.sync_copy(stage, out_hbm.at[wid])


def kernel(z_hyp, batch_indices):
    qs, ep = pl.pallas_call(
        _tc_radii_body,
        out_shape=(
            jax.ShapeDtypeStruct((8, N // 8), jnp.float32),
            jax.ShapeDtypeStruct((8, N // 8), jnp.float32),
        ),
    )(z_hyp.reshape(8, N // 8, D))

    mesh = plsc.VectorSubcoreMesh(core_axis_name="c", subcore_axis_name="s")
    out_sc = pl.kernel(
        _sc_body,
        mesh=mesh,
        out_type=jax.ShapeDtypeStruct((NW, NBINS + 1, 16), jnp.float32),
        scratch_types=[
            pltpu.VMEM((N,), jnp.float32),
            pltpu.VMEM((ROWS_PER_TEC,), jnp.float32),
            pltpu.VMEM((ROWS_PER_TEC,), jnp.int32),
            pltpu.VMEM((NBINS + 1, 16), jnp.float32),
        ],
    )(ep.reshape(N), qs.reshape(N), batch_indices[:SC_ROWS])

    n_steps = (N - SC_ROWS) // TC_CHUNK
    outb_tc, outc_tc = pl.pallas_call(
        _tc_pair_body,
        grid=(n_steps,),
        in_specs=[
            pl.BlockSpec((1, 1, TC_CHUNK),
                         lambda i: (SC_ROWS // TC_CHUNK + i, 0, 0)),
            pl.BlockSpec((8, N // 8), lambda i: (0, 0)),
            pl.BlockSpec((1, 1, TC_CHUNK),
                         lambda i: (SC_ROWS // TC_CHUNK + i, 0, 0)),
        ],
        out_specs=(
            pl.BlockSpec((1, 1), lambda i: (0, 0)),
            pl.BlockSpec((1, 16), lambda i: (0, 0)),
        ),
        out_shape=(
            jax.ShapeDtypeStruct((1, 1), jnp.float32),
            jax.ShapeDtypeStruct((1, 16), jnp.float32),
        ),
    )(qs.reshape(8, 1, N // 8), ep, batch_indices.reshape(8, 1, N // 8))

    b_total = jnp.sum(out_sc[:, NBINS, :]) + outb_tc[0, 0]
    c = jnp.sum(out_sc[:, :NBINS, :], axis=(0, 2)) + outc_tc[0, :NBINS]
    a_idx = jnp.arange(NBINS, dtype=jnp.float32)
    sv = jnp.sum(a_idx * c)
    csq = jnp.sum(c * c)
    a_sum = 0.5 * jnp.sum(c[:, None] * c[None, :]
                          * jnp.abs(a_idx[:, None] - a_idx[None, :]))
    denom = jnp.maximum(float(N * N) - csq, 1.0)
    num = 2.0 * b_total - float(N) * sv + a_sum
    return num / denom


# speedup vs baseline: 1.1249x; 1.0881x over previous
import jax
import jax.numpy as jnp
from jax import lax
from jax.experimental import pallas as pl
from jax.experimental.pallas import tpu as pltpu, tpu_sc as plsc

TEMP_INV = 10.0
N = 2048
D = 128
NC = 2
NS = 16
NW = NC * NS
SC_ROWS = 1024
ROWS_PER_TEC = SC_ROWS // NW
TC_CHUNK = 256
NBINS = 13

INV3 = -1431655765
LIM3 = 0x55555555


def _valuation_sc(m):
    v = jnp.zeros((16,), jnp.float32)
    for _ in range(NBINS):
        t = m * INV3
        div = (m > 0) & (t >= 0) & (t <= LIM3)
        v = v + jnp.where(div, 1.0, 0.0)
        m = jnp.where(div, t, m)
    return v


def _tc_radii_body(z_ref, qs_ref, ep_ref):
    x = z_ref[...]
    q = jnp.sqrt(jnp.sum(x * x, axis=-1)) * TEMP_INV
    qs = q - 0.5 * (jnp.max(q) + jnp.min(q))
    qs = jnp.clip(qs, -87.0, 87.0)
    qs_ref[...] = qs
    ep_ref[...] = jnp.exp(qs)


def _tc_pair_body(qs_ref, ep_ref, bidx_ref, outb_ref, outc_ref):
    en = jnp.exp(-qs_ref[...])
    ep = ep_ref[...].reshape(1, N)
    sig = 1.0 / (1.0 + en.reshape(TC_CHUNK, 1) * ep)
    g = jnp.sum(sig, axis=-1)

    m = bidx_ref[...].reshape(TC_CHUNK)
    v = jnp.zeros((TC_CHUNK,), jnp.float32)
    for _ in range(NBINS):
        div = (m > 0) & (m % 3 == 0)
        v = v + div.astype(jnp.float32)
        m = jnp.where(div, m // 3, m)

    b_part = jnp.sum(v * g)
    lane = lax.broadcasted_iota(jnp.int32, (1, 16), 1)
    cnt = jnp.zeros((1, 16), jnp.float32)
    for a in range(NBINS):
        cnt = cnt + jnp.where(lane == a, jnp.sum(
            jnp.where(v == float(a), 1.0, 0.0)), 0.0)

    @pl.when(pl.program_id(0) == 0)
    def _():
        outb_ref[...] = jnp.zeros((1, 1), jnp.float32)
        outc_ref[...] = jnp.zeros((1, 16), jnp.float32)

    outb_ref[...] += b_part.reshape(1, 1)
    outc_ref[...] += cnt


def _sc_body(ep_hbm, qs_hbm, bidx_hbm, out_hbm, etab, qsb, idxb, stage):
    cid = lax.axis_index("c")
    sid = lax.axis_index("s")
    wid = cid * NS + sid

    pltpu.sync_copy(ep_hbm, etab)
    pltpu.sync_copy(
        qs_hbm.at[lax.shift_right_logical(wid, 3),
                  pl.ds((wid & 7) * ROWS_PER_TEC, ROWS_PER_TEC)],
        qsb)
    pltpu.sync_copy(bidx_hbm.at[pl.ds(wid * ROWS_PER_TEC, ROWS_PER_TEC)], idxb)

    vvecs = [_valuation_sc(idxb[pl.ds(k * 16, 16)]) for k in range(2)]
    evecs = [jnp.exp(-qsb[pl.ds(k * 16, 16)]) for k in range(2)]

    one = jnp.ones((16,), jnp.float32)

    def jstep(t, accs):
        ejv = etab[lax.shift_right_logical(t, 4),
                   pl.ds((t & 15) * 16, 16)]
        for dj in range(16):
            ej = jnp.full((16,), ejv[dj])
            accs = tuple(
                acc + one / (one + ej * ei)
                for acc, ei in zip(accs, evecs)
            )
        return accs

    zero = jnp.zeros((16,), jnp.float32)
    accs = lax.fori_loop(0, N // 16, jstep, (zero, zero))

    for a in range(NBINS):
        cnt = zero
        for vk in vvecs:
            cnt = cnt + jnp.where(vk == float(a), 1.0, 0.0)
        stage[a, :] = cnt
    bvec = zero
    for vk, acck in zip(vvecs, accs):
        bvec = bvec + vk * acck
    stage[NBINS, :] = bvec
    pltpu.sync_copy(stage, out_hbm.at[wid])


def kernel(z_hyp, batch_indices):
    qs, ep = pl.pallas_call(
        _tc_radii_body,
        out_shape=(
            jax.ShapeDtypeStruct((8, N // 8), jnp.float32),
            jax.ShapeDtypeStruct((8, N // 8), jnp.float32),
        ),
    )(z_hyp.reshape(8, N // 8, D))

    mesh = plsc.VectorSubcoreMesh(core_axis_name="c", subcore_axis_name="s")
    out_sc = pl.kernel(
        _sc_body,
        mesh=mesh,
        out_type=jax.ShapeDtypeStruct((NW, NBINS + 1, 16), jnp.float32),
        scratch_types=[
            pltpu.VMEM((8, N // 8), jnp.float32),
            pltpu.VMEM((ROWS_PER_TEC,), jnp.float32),
            pltpu.VMEM((ROWS_PER_TEC,), jnp.int32),
            pltpu.VMEM((NBINS + 1, 16), jnp.float32),
        ],
    )(ep, qs, batch_indices)

    n_steps = (N - SC_ROWS) // TC_CHUNK
    outb_tc, outc_tc = pl.pallas_call(
        _tc_pair_body,
        grid=(n_steps,),
        in_specs=[
            pl.BlockSpec((1, 1, TC_CHUNK),
                         lambda i: (SC_ROWS // TC_CHUNK + i, 0, 0)),
            pl.BlockSpec((8, N // 8), lambda i: (0, 0)),
            pl.BlockSpec((1, 1, TC_CHUNK),
                         lambda i: (SC_ROWS // TC_CHUNK + i, 0, 0)),
        ],
        out_specs=(
            pl.BlockSpec((1, 1), lambda i: (0, 0)),
            pl.BlockSpec((1, 16), lambda i: (0, 0)),
        ),
        out_shape=(
            jax.ShapeDtypeStruct((1, 1), jnp.float32),
            jax.ShapeDtypeStruct((1, 16), jnp.float32),
        ),
    )(qs.reshape(8, 1, N // 8), ep, batch_indices.reshape(8, 1, N // 8))

    b_total = jnp.sum(out_sc[:, NBINS, :]) + outb_tc[0, 0]
    c = jnp.sum(out_sc[:, :NBINS, :], axis=(0, 2)) + outc_tc[0, :NBINS]
    a_idx = jnp.arange(NBINS, dtype=jnp.float32)
    sv = jnp.sum(a_idx * c)
    csq = jnp.sum(c * c)
    a_sum = 0.5 * jnp.sum(c[:, None] * c[None, :]
                          * jnp.abs(a_idx[:, None] - a_idx[None, :]))
    denom = jnp.maximum(float(N * N) - csq, 1.0)
    num = 2.0 * b_total - float(N) * sv + a_sum
    return num / denom
